# baseline (speedup 1.0000x reference)
import functools

import jax
import jax.numpy as jnp
import numpy as np
from jax.experimental import pallas as pl
from jax.experimental.pallas import tpu as pltpu


def _sel_matrix(nchan: int, width: int) -> np.ndarray:
    half = width // 2
    s = np.zeros((nchan * width, nchan * half), dtype=np.float32)
    for c in range(nchan):
        for p in range(half):
            s[c * width + 2 * p, c * half + p] = 1.0
    return s


_SEL1 = _sel_matrix(6, 28)
_SEL2 = _sel_matrix(16, 10)


def _band_eyes(isize: int, osize: int) -> np.ndarray:
    return np.stack([np.eye(isize, osize, -d, dtype=np.float32)
                     for d in range(5)])


_S1 = _band_eyes(32, 28)
_S2 = _band_eyes(14, 10)


def _net_kernel(x_ref, w1_ref, b1_ref, w2_ref, b2_ref, sel1_ref, sel2_ref,
                wf1_ref, bf1_ref, wf2_ref, bf2_ref, gw_ref, ew_ref, eb_ref,
                w4_ref, b4_ref, out_ref, *, blk: int):
    B = blk
    bf16 = jnp.bfloat16
    X = x_ref[...]

    x5 = jnp.concatenate([X[dy:dy + 28].reshape(28 * B, 96)
                          for dy in range(5)], axis=-1)
    o1 = jnp.dot(x5, w1_ref[...], preferred_element_type=jnp.float32)

    p = jnp.max(o1.reshape(14, 2, B, 168), axis=1)
    m = jnp.maximum(p, jnp.roll(p, -1, axis=-1))
    m = jnp.maximum(m + b1_ref[...], 0.0).astype(bf16)
    p1 = jnp.dot(m.reshape(14 * B, 168), sel1_ref[...],
                 preferred_element_type=jnp.float32)
    p1 = p1.astype(bf16).reshape(14, B, 84)

    p15 = jnp.concatenate([p1[dy:dy + 10].reshape(10 * B, 84)
                           for dy in range(5)], axis=-1)
    o2 = jnp.dot(p15, w2_ref[...], preferred_element_type=jnp.float32)

    p = jnp.max(o2.reshape(5, 2, B, 160), axis=1)
    m = jnp.maximum(p, jnp.roll(p, -1, axis=-1))
    m = jnp.maximum(m + b2_ref[...], 0.0).astype(bf16)
    p2 = jnp.dot(m.reshape(5 * B, 160), sel2_ref[...],
                 preferred_element_type=jnp.float32)
    p2 = p2.astype(bf16).reshape(5, B, 80)

    p2f = jnp.concatenate([p2[py] for py in range(5)], axis=-1)
    h1 = jnp.dot(p2f, wf1_ref[...], preferred_element_type=jnp.float32)
    h1 = jnp.maximum(h1 + bf1_ref[...], 0.0)

    h2 = jnp.maximum(jnp.dot(h1.astype(bf16), wf2_ref[...],
                             preferred_element_type=jnp.float32)
                     + bf2_ref[...], 0.0)

    h2b = h2.astype(bf16)
    logits = jnp.dot(h2b, gw_ref[...],
                     preferred_element_type=jnp.float32)
    mx = jnp.max(logits, axis=-1, keepdims=True)
    e = jnp.exp(logits - mx)
    g = e / jnp.sum(e, axis=-1, keepdims=True)
    denom = jnp.sum(g, axis=-1, keepdims=True) + 1e-9

    e0 = jnp.dot(h2b, ew_ref[0], preferred_element_type=jnp.float32) \
        + eb_ref[0:1, :]
    e1 = jnp.dot(h2b, ew_ref[1], preferred_element_type=jnp.float32) \
        + eb_ref[1:2, :]
    hm = (g[:, 0:1] * e0 + g[:, 1:2] * e1) / denom

    out_ref[...] = jnp.dot(hm.astype(bf16), w4_ref[...],
                           preferred_element_type=jnp.float32) + b4_ref[...]


@functools.partial(jax.jit, static_argnames=())
def kernel(x, conv1_w, conv1_b, conv2_w, conv2_b, fc1_w, fc1_b, fc2_w, fc2_b,
           gate_w, expert_w, expert_b, fc4_w, fc4_b):
    N = x.shape[0]
    BLK = 256
    grid = N // BLK

    bf16 = jnp.bfloat16
    xt = x.transpose(2, 0, 1, 3).reshape(32, N, 96).astype(bf16)

    w1s = jnp.einsum('ocyd,dij->ycioj', conv1_w,
                     jnp.asarray(_S1)).reshape(480, 168).astype(bf16)
    w2s = jnp.einsum('ocyd,dij->ycioj', conv2_w,
                     jnp.asarray(_S2)).reshape(420, 160).astype(bf16)
    b1v = jnp.repeat(conv1_b, 28).reshape(1, 168)
    b2v = jnp.repeat(conv2_b, 10).reshape(1, 160)

    wf1 = fc1_w.reshape(120, 16, 5, 5).transpose(2, 1, 3, 0)
    wf1 = wf1.reshape(400, 120).astype(bf16)
    ew = jnp.transpose(expert_w, (0, 2, 1)).astype(bf16)

    out = pl.pallas_call(
        functools.partial(_net_kernel, blk=BLK),
        grid=(grid,),
        in_specs=[
            pl.BlockSpec((32, BLK, 96), lambda i: (0, i, 0)),
            pl.BlockSpec((480, 168), lambda i: (0, 0)),
            pl.BlockSpec((1, 168), lambda i: (0, 0)),
            pl.BlockSpec((420, 160), lambda i: (0, 0)),
            pl.BlockSpec((1, 160), lambda i: (0, 0)),
            pl.BlockSpec((168, 84), lambda i: (0, 0)),
            pl.BlockSpec((160, 80), lambda i: (0, 0)),
            pl.BlockSpec((400, 120), lambda i: (0, 0)),
            pl.BlockSpec((1, 120), lambda i: (0, 0)),
            pl.BlockSpec((120, 84), lambda i: (0, 0)),
            pl.BlockSpec((1, 84), lambda i: (0, 0)),
            pl.BlockSpec((84, 2), lambda i: (0, 0)),
            pl.BlockSpec((2, 84, 84), lambda i: (0, 0, 0)),
            pl.BlockSpec((2, 84), lambda i: (0, 0)),
            pl.BlockSpec((84, 10), lambda i: (0, 0)),
            pl.BlockSpec((1, 10), lambda i: (0, 0)),
        ],
        out_specs=pl.BlockSpec((BLK, 10), lambda i: (i, 0)),
        out_shape=jax.ShapeDtypeStruct((N, 10), jnp.float32),
        compiler_params=pltpu.CompilerParams(
            dimension_semantics=("parallel",)),
    )(xt, w1s, b1v, w2s, b2v,
      jnp.asarray(_SEL1, dtype=bf16), jnp.asarray(_SEL2, dtype=bf16),
      wf1, fc1_b.reshape(1, 120), fc2_w.T.astype(bf16),
      fc2_b.reshape(1, 84), gate_w.astype(bf16), ew, expert_b,
      fc4_w.T.astype(bf16), fc4_b.reshape(1, 10))
    return out


# speedup vs baseline: 1.0464x; 1.0464x over previous
import functools

import jax
import jax.numpy as jnp
import numpy as np
from jax.experimental import pallas as pl


def _sel_matrix(nchan: int, width: int) -> np.ndarray:
    half = width // 2
    s = np.zeros((nchan * width, nchan * half), dtype=np.float32)
    for c in range(nchan):
        for p in range(half):
            s[c * width + 2 * p, c * half + p] = 1.0
    return s


_SEL1 = _sel_matrix(6, 28)
_SEL2 = _sel_matrix(16, 10)


def _band_eyes(isize: int, osize: int) -> np.ndarray:
    return np.stack([np.eye(isize, osize, -d, dtype=np.float32)
                     for d in range(5)])


_S1 = _band_eyes(32, 28)
_S2 = _band_eyes(14, 10)


def _net_kernel(x_ref, w1_ref, b1_ref, w2_ref, b2_ref, sel1_ref, sel2_ref,
                wf1_ref, bf1_ref, wf2_ref, bf2_ref, gw_ref, ew_ref, eb_ref,
                w4_ref, b4_ref, out_ref, *, blk: int):
    B = blk
    bf16 = jnp.bfloat16
    X = x_ref[...]

    x5 = jnp.concatenate([X[dy:dy + 28].reshape(28 * B, 96)
                          for dy in range(5)], axis=-1)
    o1 = jnp.dot(x5, w1_ref[...], preferred_element_type=jnp.float32)

    p = jnp.max(o1.reshape(14, 2, B, 168), axis=1)
    m = jnp.maximum(p, jnp.roll(p, -1, axis=-1))
    m = jnp.maximum(m + b1_ref[...], 0.0).astype(bf16)
    p1 = jnp.dot(m.reshape(14 * B, 168), sel1_ref[...],
                 preferred_element_type=jnp.float32)
    p1 = p1.astype(bf16).reshape(14, B, 84)

    p15 = jnp.concatenate([p1[dy:dy + 10].reshape(10 * B, 84)
                           for dy in range(5)], axis=-1)
    o2 = jnp.dot(p15, w2_ref[...], preferred_element_type=jnp.float32)

    p = jnp.max(o2.reshape(5, 2, B, 160), axis=1)
    m = jnp.maximum(p, jnp.roll(p, -1, axis=-1))
    m = jnp.maximum(m + b2_ref[...], 0.0).astype(bf16)
    p2 = jnp.dot(m.reshape(5 * B, 160), sel2_ref[...],
                 preferred_element_type=jnp.float32)
    p2 = p2.astype(bf16).reshape(5, B, 80)

    p2f = jnp.concatenate([p2[py] for py in range(5)], axis=-1)
    h1 = jnp.dot(p2f, wf1_ref[...], preferred_element_type=jnp.float32)
    h1 = jnp.maximum(h1 + bf1_ref[...], 0.0)

    h2 = jnp.maximum(jnp.dot(h1.astype(bf16), wf2_ref[...],
                             preferred_element_type=jnp.float32)
                     + bf2_ref[...], 0.0)

    h2b = h2.astype(bf16)
    logits = jnp.dot(h2b, gw_ref[...],
                     preferred_element_type=jnp.float32)
    mx = jnp.max(logits, axis=-1, keepdims=True)
    e = jnp.exp(logits - mx)
    g = e / jnp.sum(e, axis=-1, keepdims=True)
    denom = jnp.sum(g, axis=-1, keepdims=True) + 1e-9

    e0 = jnp.dot(h2b, ew_ref[0], preferred_element_type=jnp.float32) \
        + eb_ref[0:1, :]
    e1 = jnp.dot(h2b, ew_ref[1], preferred_element_type=jnp.float32) \
        + eb_ref[1:2, :]
    hm = (g[:, 0:1] * e0 + g[:, 1:2] * e1) / denom

    out_ref[...] = jnp.dot(hm.astype(bf16), w4_ref[...],
                           preferred_element_type=jnp.float32) + b4_ref[...]


@functools.partial(jax.jit, static_argnames=())
def kernel(x, conv1_w, conv1_b, conv2_w, conv2_b, fc1_w, fc1_b, fc2_w, fc2_b,
           gate_w, expert_w, expert_b, fc4_w, fc4_b):
    N = x.shape[0]
    BLK = 512
    grid = N // BLK

    bf16 = jnp.bfloat16
    xt = x.transpose(2, 0, 1, 3).reshape(32, N, 96).astype(bf16)

    w1s = jnp.einsum('ocyd,dij->ycioj', conv1_w,
                     jnp.asarray(_S1)).reshape(480, 168).astype(bf16)
    w2s = jnp.einsum('ocyd,dij->ycioj', conv2_w,
                     jnp.asarray(_S2)).reshape(420, 160).astype(bf16)
    b1v = jnp.repeat(conv1_b, 28).reshape(1, 168)
    b2v = jnp.repeat(conv2_b, 10).reshape(1, 160)

    wf1 = fc1_w.reshape(120, 16, 5, 5).transpose(2, 1, 3, 0)
    wf1 = wf1.reshape(400, 120).astype(bf16)
    ew = jnp.transpose(expert_w, (0, 2, 1)).astype(bf16)

    out = pl.pallas_call(
        functools.partial(_net_kernel, blk=BLK),
        grid=(grid,),
        in_specs=[
            pl.BlockSpec((32, BLK, 96), lambda i: (0, i, 0)),
            pl.BlockSpec((480, 168), lambda i: (0, 0)),
            pl.BlockSpec((1, 168), lambda i: (0, 0)),
            pl.BlockSpec((420, 160), lambda i: (0, 0)),
            pl.BlockSpec((1, 160), lambda i: (0, 0)),
            pl.BlockSpec((168, 84), lambda i: (0, 0)),
            pl.BlockSpec((160, 80), lambda i: (0, 0)),
            pl.BlockSpec((400, 120), lambda i: (0, 0)),
            pl.BlockSpec((1, 120), lambda i: (0, 0)),
            pl.BlockSpec((120, 84), lambda i: (0, 0)),
            pl.BlockSpec((1, 84), lambda i: (0, 0)),
            pl.BlockSpec((84, 2), lambda i: (0, 0)),
            pl.BlockSpec((2, 84, 84), lambda i: (0, 0, 0)),
            pl.BlockSpec((2, 84), lambda i: (0, 0)),
            pl.BlockSpec((84, 10), lambda i: (0, 0)),
            pl.BlockSpec((1, 10), lambda i: (0, 0)),
        ],
        out_specs=pl.BlockSpec((BLK, 10), lambda i: (i, 0)),
        out_shape=jax.ShapeDtypeStruct((N, 10), jnp.float32),
    )(xt, w1s, b1v, w2s, b2v,
      jnp.asarray(_SEL1, dtype=bf16), jnp.asarray(_SEL2, dtype=bf16),
      wf1, fc1_b.reshape(1, 120), fc2_w.T.astype(bf16),
      fc2_b.reshape(1, 84), gate_w.astype(bf16), ew, expert_b,
      fc4_w.T.astype(bf16), fc4_b.reshape(1, 10))
    return out


# speedup vs baseline: 1.0678x; 1.0204x over previous
import functools

import jax
import jax.numpy as jnp
import numpy as np
from jax.experimental import pallas as pl


def _sel_matrix(nchan: int, width: int) -> np.ndarray:
    half = width // 2
    s = np.zeros((nchan * width, nchan * half), dtype=np.float32)
    for c in range(nchan):
        for p in range(half):
            s[c * width + 2 * p, c * half + p] = 1.0
    return s


_SEL1 = _sel_matrix(6, 28)
_SEL2 = _sel_matrix(16, 10)


def _band_eyes(isize: int, osize: int) -> np.ndarray:
    return np.stack([np.eye(isize, osize, -d, dtype=np.float32)
                     for d in range(5)])


_S1 = _band_eyes(32, 28)
_S2 = _band_eyes(14, 10)


def _net_kernel(x_ref, w1_ref, b1_ref, w2_ref, b2_ref, sel1_ref, sel2_ref,
                wf1_ref, bf1_ref, wf2_ref, bf2_ref, gw_ref, ew_ref, eb_ref,
                w4_ref, b4_ref, out_ref, *, blk: int):
    B = blk
    bf16 = jnp.bfloat16
    X = x_ref[...]

    x5 = jnp.concatenate([X[dy:dy + 28].reshape(28 * B, 96)
                          for dy in range(5)], axis=-1)
    o1 = jnp.dot(x5, w1_ref[...], preferred_element_type=jnp.float32)

    p = jnp.max(o1.reshape(14, 2, B, 168), axis=1)
    m = jnp.maximum(p, jnp.roll(p, -1, axis=-1))
    m = jnp.maximum(m + b1_ref[...], 0.0).astype(bf16)
    p1 = jnp.dot(m.reshape(14 * B, 168), sel1_ref[...],
                 preferred_element_type=jnp.float32)
    p1 = p1.astype(bf16).reshape(14, B, 84)

    p15 = jnp.concatenate([p1[dy:dy + 10].reshape(10 * B, 84)
                           for dy in range(5)], axis=-1)
    o2 = jnp.dot(p15, w2_ref[...], preferred_element_type=jnp.float32)

    p = jnp.max(o2.reshape(5, 2, B, 160), axis=1)
    m = jnp.maximum(p, jnp.roll(p, -1, axis=-1))
    m = jnp.maximum(m + b2_ref[...], 0.0).astype(bf16)
    p2 = jnp.dot(m.reshape(5 * B, 160), sel2_ref[...],
                 preferred_element_type=jnp.float32)
    p2 = p2.astype(bf16).reshape(5, B, 80)

    p2f = jnp.concatenate([p2[py] for py in range(5)], axis=-1)
    h1 = jnp.dot(p2f, wf1_ref[...], preferred_element_type=jnp.float32)
    h1 = jnp.maximum(h1 + bf1_ref[...], 0.0)

    h2 = jnp.maximum(jnp.dot(h1.astype(bf16), wf2_ref[...],
                             preferred_element_type=jnp.float32)
                     + bf2_ref[...], 0.0)

    h2b = h2.astype(bf16)
    logits = jnp.dot(h2b, gw_ref[...],
                     preferred_element_type=jnp.float32)
    mx = jnp.max(logits, axis=-1, keepdims=True)
    e = jnp.exp(logits - mx)
    g = e / jnp.sum(e, axis=-1, keepdims=True)
    denom = jnp.sum(g, axis=-1, keepdims=True) + 1e-9

    e0 = jnp.dot(h2b, ew_ref[0], preferred_element_type=jnp.float32) \
        + eb_ref[0:1, :]
    e1 = jnp.dot(h2b, ew_ref[1], preferred_element_type=jnp.float32) \
        + eb_ref[1:2, :]
    hm = (g[:, 0:1] * e0 + g[:, 1:2] * e1) / denom

    out_ref[...] = jnp.dot(hm.astype(bf16), w4_ref[...],
                           preferred_element_type=jnp.float32) + b4_ref[...]


@functools.partial(jax.jit, static_argnames=())
def kernel(x, conv1_w, conv1_b, conv2_w, conv2_b, fc1_w, fc1_b, fc2_w, fc2_b,
           gate_w, expert_w, expert_b, fc4_w, fc4_b):
    N = x.shape[0]
    BLK = 1024
    grid = N // BLK

    bf16 = jnp.bfloat16
    xt = x.transpose(2, 0, 1, 3).reshape(32, N, 96).astype(bf16)

    w1s = jnp.einsum('ocyd,dij->ycioj', conv1_w,
                     jnp.asarray(_S1)).reshape(480, 168).astype(bf16)
    w2s = jnp.einsum('ocyd,dij->ycioj', conv2_w,
                     jnp.asarray(_S2)).reshape(420, 160).astype(bf16)
    b1v = jnp.repeat(conv1_b, 28).reshape(1, 168)
    b2v = jnp.repeat(conv2_b, 10).reshape(1, 160)

    wf1 = fc1_w.reshape(120, 16, 5, 5).transpose(2, 1, 3, 0)
    wf1 = wf1.reshape(400, 120).astype(bf16)
    ew = jnp.transpose(expert_w, (0, 2, 1)).astype(bf16)

    out = pl.pallas_call(
        functools.partial(_net_kernel, blk=BLK),
        grid=(grid,),
        in_specs=[
            pl.BlockSpec((32, BLK, 96), lambda i: (0, i, 0)),
            pl.BlockSpec((480, 168), lambda i: (0, 0)),
            pl.BlockSpec((1, 168), lambda i: (0, 0)),
            pl.BlockSpec((420, 160), lambda i: (0, 0)),
            pl.BlockSpec((1, 160), lambda i: (0, 0)),
            pl.BlockSpec((168, 84), lambda i: (0, 0)),
            pl.BlockSpec((160, 80), lambda i: (0, 0)),
            pl.BlockSpec((400, 120), lambda i: (0, 0)),
            pl.BlockSpec((1, 120), lambda i: (0, 0)),
            pl.BlockSpec((120, 84), lambda i: (0, 0)),
            pl.BlockSpec((1, 84), lambda i: (0, 0)),
            pl.BlockSpec((84, 2), lambda i: (0, 0)),
            pl.BlockSpec((2, 84, 84), lambda i: (0, 0, 0)),
            pl.BlockSpec((2, 84), lambda i: (0, 0)),
            pl.BlockSpec((84, 10), lambda i: (0, 0)),
            pl.BlockSpec((1, 10), lambda i: (0, 0)),
        ],
        out_specs=pl.BlockSpec((BLK, 10), lambda i: (i, 0)),
        out_shape=jax.ShapeDtypeStruct((N, 10), jnp.float32),
    )(xt, w1s, b1v, w2s, b2v,
      jnp.asarray(_SEL1, dtype=bf16), jnp.asarray(_SEL2, dtype=bf16),
      wf1, fc1_b.reshape(1, 120), fc2_w.T.astype(bf16),
      fc2_b.reshape(1, 84), gate_w.astype(bf16), ew, expert_b,
      fc4_w.T.astype(bf16), fc4_b.reshape(1, 10))
    return out
